# baseline (speedup 1.0000x reference)
import functools

import jax
import jax.numpy as jnp
from jax import lax
from jax.experimental import pallas as pl
from jax.experimental.pallas import tpu as pltpu
from jax.experimental.pallas import tpu_sc as plsc


def _pose_emb_sc(pose_W, p, b):
    n, d = pose_W.shape
    info = plsc.get_sparse_core_info()
    nw = info.num_cores * info.num_subcores
    assert b == nw
    mesh = plsc.VectorSubcoreMesh(core_axis_name="c", subcore_axis_name="s")

    @functools.partial(
        pl.kernel, mesh=mesh,
        out_type=jax.ShapeDtypeStruct((b, 2 * d), jnp.float32),
        scratch_types=[
            pltpu.VMEM((1,), jnp.int32),
            pltpu.VMEM((1, d), jnp.float32),
            pltpu.SemaphoreType.DMA,
        ],
    )
    def k(pose_hbm, p_hbm, out_hbm, idx_v, row_v, sem):
        wid = lax.axis_index("s") * info.num_cores + lax.axis_index("c")
        pltpu.sync_copy(p_hbm, idx_v)
        pltpu.async_copy(pose_hbm.at[idx_v], row_v, sem).wait()
        pltpu.sync_copy(row_v.at[0], out_hbm.at[wid, pl.ds(0, d)])
        pltpu.sync_copy(row_v.at[0], out_hbm.at[wid, pl.ds(d, d)])

    return k(pose_W, jnp.reshape(p, (1,)).astype(jnp.int32))


def kernel(x, row_W, col_W, pose_W, p):
    b, _, h, w = x.shape
    d = row_W.shape[1]

    p_emb = _pose_emb_sc(pose_W, p, b)

    group = 4
    nsem = 8

    def body(row_ref, col_ref, mt_ref, scratch, sem):
        col_s = col_ref[1:w + 1, :]
        row_s = row_ref[1:h + 1, :]
        bc_col = jnp.broadcast_to(col_s[None, None, :, :], (group, h, w, d))
        bc_row = jnp.broadcast_to(row_s[None, :, None, :], (group, h, w, d))
        scratch[...] = jnp.concatenate([bc_col, bc_row], axis=-1)
        copies = [
            pltpu.make_async_copy(
                scratch, mt_ref.at[pl.ds(i * group, group)],
                sem.at[i % nsem])
            for i in range(b // group)
        ]
        for c in copies:
            c.start()
        for c in copies:
            c.wait()

    m_t = pl.pallas_call(
        body,
        grid=(1,),
        in_specs=[
            pl.BlockSpec(row_W.shape, lambda i: (0, 0)),
            pl.BlockSpec(col_W.shape, lambda i: (0, 0)),
        ],
        out_specs=pl.BlockSpec(memory_space=pl.ANY),
        out_shape=jax.ShapeDtypeStruct((b, h, w, 2 * d), jnp.float32),
        scratch_shapes=[
            pltpu.VMEM((group, h, w, 2 * d), jnp.float32),
            pltpu.SemaphoreType.DMA((nsem,)),
        ],
    )(row_W, col_W)

    return (p_emb, jnp.transpose(m_t, (0, 3, 1, 2)))


# speedup vs baseline: 2.0640x; 2.0640x over previous
import jax
import jax.numpy as jnp
from jax.experimental import pallas as pl
from jax.experimental.pallas import tpu as pltpu


def kernel(x, row_W, col_W, pose_W, p):
    b, _, h, w = x.shape
    d = row_W.shape[1]

    group = 4
    nsem = 8

    def body(p_ref, row_ref, col_ref, pose_ref, mt_ref, pemb_ref,
             scratch, sem):
        col_s = col_ref[1:w + 1, :]
        row_s = row_ref[1:h + 1, :]
        bc_col = jnp.broadcast_to(col_s[None, None, :, :], (group, h, w, d))
        bc_row = jnp.broadcast_to(row_s[None, :, None, :], (group, h, w, d))
        scratch[...] = jnp.concatenate([bc_col, bc_row], axis=-1)
        half = jnp.broadcast_to(pose_ref[p_ref[0], :][None, :], (b, d))
        pemb_ref[...] = jnp.concatenate([half, half], axis=1)
        copies = [
            pltpu.make_async_copy(
                scratch, mt_ref.at[pl.ds(i * group, group)],
                sem.at[i % nsem])
            for i in range(b // group)
        ]
        for c in copies:
            c.start()
        for c in copies:
            c.wait()

    grid_spec = pltpu.PrefetchScalarGridSpec(
        num_scalar_prefetch=1,
        grid=(1,),
        in_specs=[
            pl.BlockSpec(row_W.shape, lambda i, p_: (0, 0)),
            pl.BlockSpec(col_W.shape, lambda i, p_: (0, 0)),
            pl.BlockSpec(pose_W.shape, lambda i, p_: (0, 0)),
        ],
        out_specs=[
            pl.BlockSpec(memory_space=pl.ANY),
            pl.BlockSpec((b, 2 * d), lambda i, p_: (0, 0)),
        ],
        scratch_shapes=[
            pltpu.VMEM((group, h, w, 2 * d), jnp.float32),
            pltpu.SemaphoreType.DMA((nsem,)),
        ],
    )
    m_t, p_emb = pl.pallas_call(
        body,
        grid_spec=grid_spec,
        out_shape=[
            jax.ShapeDtypeStruct((b, h, w, 2 * d), jnp.float32),
            jax.ShapeDtypeStruct((b, 2 * d), jnp.float32),
        ],
    )(jnp.reshape(p, (1,)).astype(jnp.int32), row_W, col_W, pose_W)

    return (p_emb, jnp.transpose(m_t, (0, 3, 1, 2)))


# speedup vs baseline: 2.0824x; 1.0089x over previous
import jax
import jax.numpy as jnp
from jax.experimental import pallas as pl
from jax.experimental.pallas import tpu as pltpu


def kernel(x, row_W, col_W, pose_W, p):
    b, _, h, w = x.shape
    d = row_W.shape[1]

    group = 1
    nsem = 1

    def body(p_ref, row_ref, col_ref, pose_ref, mt_ref, pemb_ref,
             scratch, sem):
        col_s = col_ref[1:w + 1, :]
        row_s = row_ref[1:h + 1, :]
        bc_col = jnp.broadcast_to(col_s[None, None, :, :], (group, h, w, d))
        bc_row = jnp.broadcast_to(row_s[None, :, None, :], (group, h, w, d))
        scratch[...] = jnp.concatenate([bc_col, bc_row], axis=-1)
        half = jnp.broadcast_to(pose_ref[p_ref[0], :][None, :], (b, d))
        pemb_ref[...] = jnp.concatenate([half, half], axis=1)
        copies = [
            pltpu.make_async_copy(
                scratch, mt_ref.at[pl.ds(i * group, group)],
                sem.at[i % nsem])
            for i in range(b // group)
        ]
        for c in copies:
            c.start()
        for c in copies:
            c.wait()

    grid_spec = pltpu.PrefetchScalarGridSpec(
        num_scalar_prefetch=1,
        grid=(1,),
        in_specs=[
            pl.BlockSpec(row_W.shape, lambda i, p_: (0, 0)),
            pl.BlockSpec(col_W.shape, lambda i, p_: (0, 0)),
            pl.BlockSpec(pose_W.shape, lambda i, p_: (0, 0)),
        ],
        out_specs=[
            pl.BlockSpec(memory_space=pl.ANY),
            pl.BlockSpec((b, 2 * d), lambda i, p_: (0, 0)),
        ],
        scratch_shapes=[
            pltpu.VMEM((group, h, w, 2 * d), jnp.float32),
            pltpu.SemaphoreType.DMA((nsem,)),
        ],
    )
    m_t, p_emb = pl.pallas_call(
        body,
        grid_spec=grid_spec,
        out_shape=[
            jax.ShapeDtypeStruct((b, h, w, 2 * d), jnp.float32),
            jax.ShapeDtypeStruct((b, 2 * d), jnp.float32),
        ],
    )(jnp.reshape(p, (1,)).astype(jnp.int32), row_W, col_W, pose_W)

    return (p_emb, jnp.transpose(m_t, (0, 3, 1, 2)))
